# Initial kernel scaffold; baseline (speedup 1.0000x reference)
import jax
import jax.numpy as jnp
from jax.experimental import pallas as pl


def kernel(x, edge_index, steps, W_msg, b_msg, W_out, b_out):
    raise NotImplementedError("write your pallas kernel here")


# speedup vs baseline: 5.1943x; 5.1943x over previous
import functools

import jax
import jax.numpy as jnp
from jax import lax
from jax.experimental import pallas as pl
from jax.experimental.pallas import tpu as pltpu, tpu_sc as plsc

N_SPATIAL = 3
N_POL = 8
DT = 0.01

N = 10000
E = 320000
D = 128

NC = 2
NS = 16
NW = NC * NS
EB = 128
NB = -(-E // (NW * EB))
EPT = NB * EB
EPAD = NW * EPT
ACC_ROWS = 10240
ZCHUNK = ACC_ROWS // NS // 128
OUT_RPT = 640
OUT_TAIL = N - 15 * OUT_RPT

_RBLK = 2000
_GRID = N // _RBLK


def _tc_pre_body(x_ref, wm_ref, bm_ref, r_ref):
    g = jnp.dot(x_ref[...], wm_ref[...], preferred_element_type=jnp.float32)
    r_ref[...] = jnp.maximum(g + bm_ref[...], 0.0)


def _tc_pre(x, W_msg, b_msg2):
    return pl.pallas_call(
        _tc_pre_body,
        grid=(_GRID,),
        in_specs=[
            pl.BlockSpec((_RBLK, D), lambda i: (i, 0)),
            pl.BlockSpec((D, D), lambda i: (0, 0)),
            pl.BlockSpec((1, D), lambda i: (0, 0)),
        ],
        out_specs=pl.BlockSpec((_RBLK, D), lambda i: (i, 0)),
        out_shape=jax.ShapeDtypeStruct((N, D), jnp.float32),
    )(x, W_msg, b_msg2)


def _tc_update_body(x_ref, m_ref, wo_ref, bo_ref, wm_ref, bm_ref, xo_ref, ro_ref):
    msgs = m_ref[0] + m_ref[1]
    out = jnp.dot(msgs, wo_ref[...], preferred_element_type=jnp.float32) + bo_ref[...]
    xn = x_ref[...] + DT * out
    col = lax.broadcasted_iota(jnp.int32, (_RBLK, D), 1)
    pm = (col >= N_SPATIAL) & (col < N_SPATIAL + 2 * N_POL)
    sq = jnp.where(pm, xn * xn, 0.0)
    nrm = jnp.sqrt(jnp.sum(sq, axis=1, keepdims=True))
    scale = 1.0 / jnp.maximum(nrm, 1e-8)
    xn = jnp.where(pm, xn * scale, xn)
    xo_ref[...] = xn
    g = jnp.dot(xn, wm_ref[...], preferred_element_type=jnp.float32)
    ro_ref[...] = jnp.maximum(g + bm_ref[...], 0.0)


def _tc_update(x, m2, W_out, b_out2, W_msg, b_msg2):
    return pl.pallas_call(
        _tc_update_body,
        grid=(_GRID,),
        in_specs=[
            pl.BlockSpec((_RBLK, D), lambda i: (i, 0)),
            pl.BlockSpec((NC, _RBLK, D), lambda i: (0, i, 0)),
            pl.BlockSpec((D, D), lambda i: (0, 0)),
            pl.BlockSpec((1, D), lambda i: (0, 0)),
            pl.BlockSpec((D, D), lambda i: (0, 0)),
            pl.BlockSpec((1, D), lambda i: (0, 0)),
        ],
        out_specs=[
            pl.BlockSpec((_RBLK, D), lambda i: (i, 0)),
            pl.BlockSpec((_RBLK, D), lambda i: (i, 0)),
        ],
        out_shape=[
            jax.ShapeDtypeStruct((N, D), jnp.float32),
            jax.ShapeDtypeStruct((N, D), jnp.float32),
        ],
    )(x, m2, W_out, b_out2, W_msg, b_msg2)


def _sc_seg_body(r_hbm, src_hbm, dst_hbm, out_hbm, idx_s, idx_d, rows, acc, sem):
    cid = lax.axis_index("c")
    sid = lax.axis_index("s")
    wid = cid * NS + sid
    pltpu.sync_copy(src_hbm.at[wid], idx_s)
    pltpu.sync_copy(dst_hbm.at[wid], idx_d)

    def _zrow(i, c):
        def _zcol(j, c2):
            rows[i, pl.ds(j * 16, 16)] = jnp.zeros((16,), jnp.float32)
            return c2
        return lax.fori_loop(0, D // 16, _zcol, c)
    lax.fori_loop(0, 128, _zrow, 0)

    def _zcopy(k, c):
        pltpu.sync_copy(rows, acc.at[pl.ds(sid * (ZCHUNK * 128) + k * 128, 128)])
        return c
    lax.fori_loop(0, ZCHUNK, _zcopy, 0)
    plsc.subcore_barrier()

    def _edge(j, c):
        pltpu.async_copy(r_hbm.at[idx_s.at[j]], rows, sem).wait()
        pltpu.sync_copy(rows, acc.at[idx_d.at[j]], add=True)
        return c
    lax.fori_loop(0, NB, _edge, 0)
    plsc.subcore_barrier()

    @pl.when(sid < NS - 1)
    def _():
        pltpu.sync_copy(acc.at[pl.ds(sid * OUT_RPT, OUT_RPT)],
                        out_hbm.at[cid, pl.ds(sid * OUT_RPT, OUT_RPT)])

    @pl.when(sid == NS - 1)
    def _():
        pltpu.sync_copy(acc.at[pl.ds((NS - 1) * OUT_RPT, OUT_TAIL)],
                        out_hbm.at[cid, pl.ds((NS - 1) * OUT_RPT, OUT_TAIL)])


@functools.lru_cache(maxsize=1)
def _sc_segsum_call():
    mesh = plsc.VectorSubcoreMesh(core_axis_name="c", subcore_axis_name="s")
    return pl.kernel(
        _sc_seg_body,
        mesh=mesh,
        out_type=jax.ShapeDtypeStruct((NC, N, D), jnp.float32),
        scratch_types=[
            pltpu.VMEM((NB, EB), jnp.int32),
            pltpu.VMEM((NB, EB), jnp.int32),
            pltpu.VMEM((EB, D), jnp.float32),
            pltpu.VMEM_SHARED((ACC_ROWS, D), jnp.float32),
            pltpu.SemaphoreType.DMA,
        ],
    )


def kernel(x, edge_index, steps, W_msg, b_msg, W_out, b_out):
    src = edge_index[0].astype(jnp.int32)
    dst = edge_index[1].astype(jnp.int32)
    npad = EPAD - E
    src3 = jnp.concatenate([src, jnp.zeros((npad,), jnp.int32)]).reshape(NW, NB, EB)
    dst3 = jnp.concatenate([dst, jnp.full((npad,), N, jnp.int32)]).reshape(NW, NB, EB)
    b_msg2 = b_msg.reshape(1, D)
    b_out2 = b_out.reshape(1, D)

    seg = _sc_segsum_call()
    r0 = _tc_pre(x, W_msg, b_msg2)

    def _step(_, carry):
        xc, rc = carry
        m2 = seg(rc, src3, dst3)
        xn, rn = _tc_update(xc, m2, W_out, b_out2, W_msg, b_msg2)
        return (xn, rn)

    xf, _ = lax.fori_loop(0, steps, _step, (x, r0))
    return xf
